# baseline (speedup 1.0000x reference)
import functools

import jax
import jax.numpy as jnp
from jax import lax
from jax.experimental import pallas as pl
from jax.experimental.pallas import tpu as pltpu
from jax.experimental.pallas import tpu_sc as plsc

_F0_MIN = 50.0
_F0_MAX = 800.0
_NUM_BINS = 256
_EMBED_DIM = 512

_NC = 2
_NS = 16
_NW = _NC * _NS

_CHUNK = 64
_SC_FRAC_NUM = 1
_SC_FRAC_DEN = 4
_BLK = 2048


def _index_body(f0_ref, idx_ref):
    log_min = jnp.log(jnp.float32(_F0_MIN))
    log_max = jnp.log(jnp.float32(_F0_MAX))
    log_range = log_max - log_min
    f0 = jnp.clip(f0_ref[...], _F0_MIN, _F0_MAX)
    f0_norm = (jnp.log(f0) - log_min) / log_range
    idx = jnp.clip(jnp.round(f0_norm * (_NUM_BINS - 1)), 0, _NUM_BINS - 1)
    idx_ref[...] = idx.astype(jnp.int32)


def _compute_indices(f0_seq):
    return pl.pallas_call(
        _index_body,
        out_shape=jax.ShapeDtypeStruct(f0_seq.shape, jnp.int32),
    )(f0_seq)


def _make_sc_gather(n_sc, d):
    tok_per_w = n_sc // _NW
    n_chunks = tok_per_w // _CHUNK
    mesh = plsc.VectorSubcoreMesh(core_axis_name="c", subcore_axis_name="s")

    @functools.partial(
        pl.kernel,
        mesh=mesh,
        out_type=jax.ShapeDtypeStruct((n_sc, d), jnp.float32),
        scratch_types=[
            pltpu.VMEM((tok_per_w,), jnp.int32),
            pltpu.VMEM((_CHUNK, d), jnp.float32),
            pltpu.SemaphoreType.DMA,
        ],
    )
    def gather(table_hbm, idx_hbm, out_hbm, idx_v, rows_v, sem):
        wid = lax.axis_index("s") * _NC + lax.axis_index("c")
        base = wid * tok_per_w
        pltpu.sync_copy(idx_hbm.at[pl.ds(base, tok_per_w)], idx_v)

        def body(k, carry):
            tok = pl.multiple_of(k * _CHUNK, _CHUNK)
            pltpu.async_copy(
                table_hbm.at[idx_v.at[pl.ds(tok, _CHUNK)]], rows_v, sem
            ).wait()
            pltpu.sync_copy(rows_v, out_hbm.at[pl.ds(base + tok, _CHUNK)])
            return carry

        lax.fori_loop(0, n_chunks, body, 0)

    return gather


def _onehot_body(f0_ref, table_ref, out_ref):
    log_min = jnp.log(jnp.float32(_F0_MIN))
    log_max = jnp.log(jnp.float32(_F0_MAX))
    log_range = log_max - log_min
    f0 = jnp.clip(f0_ref[...], _F0_MIN, _F0_MAX)
    f0_norm = (jnp.log(f0) - log_min) / log_range
    idx = jnp.clip(jnp.round(f0_norm * (_NUM_BINS - 1)), 0, _NUM_BINS - 1)
    idx = idx.astype(jnp.int32).reshape(_BLK, 1)
    bins = lax.broadcasted_iota(jnp.int32, (_BLK, _NUM_BINS), 1)
    onehot = jnp.where(bins == idx, 1.0, 0.0).astype(jnp.float32)
    out_ref[...] = jnp.dot(
        onehot, table_ref[...], preferred_element_type=jnp.float32
    )


def _tc_fill(f0_rest, embedding, d):
    n_tc = f0_rest.shape[0]
    grid = n_tc // _BLK
    return pl.pallas_call(
        _onehot_body,
        grid=(grid,),
        in_specs=[
            pl.BlockSpec((_BLK,), lambda i: (i,)),
            pl.BlockSpec((_NUM_BINS, d), lambda i: (0, 0)),
        ],
        out_specs=pl.BlockSpec((_BLK, d), lambda i: (i, 0)),
        out_shape=jax.ShapeDtypeStruct((n_tc, d), jnp.float32),
    )(f0_rest, embedding)


def kernel(f0_seq, embedding):
    b, s = f0_seq.shape
    n_tokens = b * s
    d = embedding.shape[1]
    n_sc = n_tokens * _SC_FRAC_NUM // _SC_FRAC_DEN
    f0_flat = f0_seq.reshape(n_tokens)
    idx = _compute_indices(f0_seq).reshape(n_tokens)
    sc_out = _make_sc_gather(n_sc, d)(embedding, idx)
    tc_out = _tc_fill(f0_flat[n_sc:], embedding, d)
    return jnp.concatenate([sc_out, tc_out], axis=0).reshape(b, s, d)


# speedup vs baseline: 1.8932x; 1.8932x over previous
import functools

import jax
import jax.numpy as jnp
from jax import lax
from jax.experimental import pallas as pl
from jax.experimental.pallas import tpu as pltpu
from jax.experimental.pallas import tpu_sc as plsc

_F0_MIN = 50.0
_F0_MAX = 800.0
_NUM_BINS = 256
_EMBED_DIM = 512

_NC = 2
_NS = 16
_NW = _NC * _NS

_CHUNK = 64
_SC_FRAC_NUM = 1
_SC_FRAC_DEN = 4
_BLK = 2048


def _index_body(f0_ref, idx_ref):
    log_min = jnp.log(jnp.float32(_F0_MIN))
    log_max = jnp.log(jnp.float32(_F0_MAX))
    log_range = log_max - log_min
    f0 = jnp.clip(f0_ref[...], _F0_MIN, _F0_MAX)
    f0_norm = (jnp.log(f0) - log_min) / log_range
    idx = jnp.clip(jnp.round(f0_norm * (_NUM_BINS - 1)), 0, _NUM_BINS - 1)
    idx_ref[...] = idx.astype(jnp.int32)


def _compute_indices(f0_seq):
    return pl.pallas_call(
        _index_body,
        out_shape=jax.ShapeDtypeStruct(f0_seq.shape, jnp.int32),
    )(f0_seq)


def _make_sc_gather(n_tokens, n_sc, d):
    tok_per_w = n_sc // _NW
    n_chunks = tok_per_w // _CHUNK
    mesh = plsc.VectorSubcoreMesh(core_axis_name="c", subcore_axis_name="s")

    @functools.partial(
        pl.kernel,
        mesh=mesh,
        out_type=jax.ShapeDtypeStruct((n_tokens, d), jnp.float32),
        scratch_types=[
            pltpu.VMEM((tok_per_w,), jnp.int32),
            pltpu.VMEM((_CHUNK, d), jnp.float32),
            pltpu.SemaphoreType.DMA,
        ],
    )
    def gather(table_hbm, idx_hbm, out_hbm, idx_v, rows_v, sem):
        wid = lax.axis_index("s") * _NC + lax.axis_index("c")
        base = wid * tok_per_w
        pltpu.sync_copy(idx_hbm.at[pl.ds(base, tok_per_w)], idx_v)

        def body(k, carry):
            tok = pl.multiple_of(k * _CHUNK, _CHUNK)
            pltpu.async_copy(
                table_hbm.at[idx_v.at[pl.ds(tok, _CHUNK)]], rows_v, sem
            ).wait()
            pltpu.sync_copy(rows_v, out_hbm.at[pl.ds(base + tok, _CHUNK)])
            return carry

        lax.fori_loop(0, n_chunks, body, 0)

    return gather


def _onehot_body(f0_ref, table_ref, _, out_ref):
    log_min = jnp.log(jnp.float32(_F0_MIN))
    log_max = jnp.log(jnp.float32(_F0_MAX))
    log_range = log_max - log_min
    f0 = jnp.clip(f0_ref[...], _F0_MIN, _F0_MAX)
    f0_norm = (jnp.log(f0) - log_min) / log_range
    idx = jnp.clip(jnp.round(f0_norm * (_NUM_BINS - 1)), 0, _NUM_BINS - 1)
    idx = idx.astype(jnp.int32).reshape(_BLK, 1)
    bins = lax.broadcasted_iota(jnp.int32, (_BLK, _NUM_BINS), 1)
    onehot = jnp.where(bins == idx, 1.0, 0.0).astype(jnp.float32)
    out_ref[...] = jnp.dot(
        onehot, table_ref[...], preferred_element_type=jnp.float32
    )


def _tc_fill(f0_rest, embedding, partial_out, n_sc):
    n_tokens, d = partial_out.shape
    n_tc = n_tokens - n_sc
    grid = n_tc // _BLK
    blk0 = n_sc // _BLK
    return pl.pallas_call(
        _onehot_body,
        grid=(grid,),
        in_specs=[
            pl.BlockSpec((_BLK,), lambda i: (i,)),
            pl.BlockSpec((_NUM_BINS, d), lambda i: (0, 0)),
            pl.BlockSpec(memory_space=pl.ANY),
        ],
        out_specs=pl.BlockSpec((_BLK, d), lambda i: (i + blk0, 0)),
        out_shape=jax.ShapeDtypeStruct((n_tokens, d), jnp.float32),
        input_output_aliases={2: 0},
    )(f0_rest, embedding, partial_out)


def kernel(f0_seq, embedding):
    b, s = f0_seq.shape
    n_tokens = b * s
    d = embedding.shape[1]
    n_sc = n_tokens * _SC_FRAC_NUM // _SC_FRAC_DEN
    f0_flat = f0_seq.reshape(n_tokens)
    b_sc = b * _SC_FRAC_NUM // _SC_FRAC_DEN
    idx = _compute_indices(f0_seq[:b_sc]).reshape(n_sc)
    sc_out = _make_sc_gather(n_tokens, n_sc, d)(embedding, idx)
    out_flat = _tc_fill(f0_flat[n_sc:], embedding, sc_out, n_sc)
    return out_flat.reshape(b, s, d)
